# baseline (speedup 1.0000x reference)
import jax
import jax.numpy as jnp
from jax.experimental import pallas as pl
from jax.experimental.pallas import tpu as pltpu
from jax.experimental.pallas import tpu_sc as plsc


_LANES = 16
_BH = 8


def _newton_rsqrt(r2):
    i = jax.lax.bitcast_convert_type(r2, jnp.int32)
    i = jnp.int32(0x5F3759DF) - jax.lax.shift_right_logical(i, 1)
    y = jax.lax.bitcast_convert_type(i, jnp.float32)
    half = 0.5 * r2
    for _ in range(3):
        y = y * (1.5 - half * y * y)
    return y


def _sc_body(xr_vmem, xi_vmem, b_vmem, o0_vmem, o1_vmem):
    @pl.loop(0, _BH)
    def _(r):
        @pl.loop(0, xr_vmem.shape[1], step=_LANES)
        def _(c):
            slc = (pl.ds(r, 1), pl.ds(c, _LANES))
            xr = xr_vmem.at[slc][...]
            xi = xi_vmem.at[slc][...]
            b = b_vmem.at[pl.ds(0, 1), pl.ds(c, _LANES)][...]
            r2 = xr * xr + xi * xi
            scale = jnp.maximum(1.0 + b * _newton_rsqrt(r2), 0.0)
            o0_vmem.at[slc][...] = scale * xr
            o1_vmem.at[slc][...] = scale * xi


def _sc_modrelu(x_real, x_imag, bias):
    n, c = x_real.shape
    mesh = plsc.VectorSubcoreMesh(core_axis_name="c", subcore_axis_name="s")

    @pl.kernel(
        out_type=jax.ShapeDtypeStruct((2, n, c), x_real.dtype),
        mesh=mesh,
        scratch_types=[],
    )
    def run(xr_hbm, xi_hbm, b_hbm, o_hbm):
        pltpu.emit_pipeline(
            _sc_body,
            grid=(n // _BH,),
            in_specs=[
                pl.BlockSpec((_BH, c), lambda i: (i, 0)),
                pl.BlockSpec((_BH, c), lambda i: (i, 0)),
                pl.BlockSpec((1, c), lambda i: (0, 0)),
            ],
            out_specs=[
                pl.BlockSpec((_BH, c), lambda i: (i, 0)),
                pl.BlockSpec((_BH, c), lambda i: (i, 0)),
            ],
            core_axis_name=("c", "s"),
            dimension_semantics=(pltpu.PARALLEL,),
        )(xr_hbm, xi_hbm, b_hbm, o_hbm.at[0], o_hbm.at[1])

    return run(x_real, x_imag, bias)


def _tc_block(xr_ref, xi_ref, b_ref, o_ref):
    xr = xr_ref[...]
    xi = xi_ref[...]
    b = b_ref[...]
    r2 = xr * xr + xi * xi
    inv_r = jax.lax.rsqrt(r2)
    scale = jnp.maximum(1.0 + b * inv_r, 0.0)
    scale = jnp.where(r2 > 0.0, scale, 1.0)
    o_ref[0, :, :] = scale * xr
    o_ref[1, :, :] = scale * xi


def _tc_modrelu(x_real, x_imag, bias):
    n, c = x_real.shape
    bn = 1024
    return pl.pallas_call(
        _tc_block,
        grid=(n // bn,),
        in_specs=[
            pl.BlockSpec((bn, c), lambda i: (i, 0)),
            pl.BlockSpec((bn, c), lambda i: (i, 0)),
            pl.BlockSpec((1, c), lambda i: (0, 0)),
        ],
        out_specs=pl.BlockSpec((2, bn, c), lambda i: (0, i, 0)),
        out_shape=jax.ShapeDtypeStruct((2, n, c), x_real.dtype),
    )(x_real, x_imag, bias)


def kernel(x_real, x_imag, bias):
    return _sc_modrelu(x_real, x_imag, bias)


# speedup vs baseline: 1.2058x; 1.2058x over previous
import jax
import jax.numpy as jnp
from jax.experimental import pallas as pl
from jax.experimental.pallas import tpu as pltpu
from jax.experimental.pallas import tpu_sc as plsc


_LANES = 16
_BH = 8


def _newton_rsqrt(r2):
    i = jax.lax.bitcast_convert_type(r2, jnp.int32)
    i = jnp.int32(0x5F3759DF) - jax.lax.shift_right_logical(i, 1)
    y = jax.lax.bitcast_convert_type(i, jnp.float32)
    half = 0.5 * r2
    for _ in range(2):
        y = y * (1.5 - half * y * y)
    return y


def _sc_body(xr_vmem, xi_vmem, b_vmem, o0_vmem, o1_vmem):
    @pl.loop(0, xr_vmem.shape[1], step=_LANES)
    def _(c):
        b = b_vmem.at[pl.ds(0, 1), pl.ds(c, _LANES)][...]
        for r in range(_BH):
            slc = (pl.ds(r, 1), pl.ds(c, _LANES))
            xr = xr_vmem.at[slc][...]
            xi = xi_vmem.at[slc][...]
            r2 = xr * xr + xi * xi
            scale = jnp.maximum(1.0 + b * _newton_rsqrt(r2), 0.0)
            o0_vmem.at[slc][...] = scale * xr
            o1_vmem.at[slc][...] = scale * xi


def _sc_modrelu(x_real, x_imag, bias):
    n, c = x_real.shape
    mesh = plsc.VectorSubcoreMesh(core_axis_name="c", subcore_axis_name="s")

    @pl.kernel(
        out_type=jax.ShapeDtypeStruct((2, n, c), x_real.dtype),
        mesh=mesh,
        scratch_types=[],
    )
    def run(xr_hbm, xi_hbm, b_hbm, o_hbm):
        pltpu.emit_pipeline(
            _sc_body,
            grid=(n // _BH,),
            in_specs=[
                pl.BlockSpec((_BH, c), lambda i: (i, 0)),
                pl.BlockSpec((_BH, c), lambda i: (i, 0)),
                pl.BlockSpec((1, c), lambda i: (0, 0)),
            ],
            out_specs=[
                pl.BlockSpec((_BH, c), lambda i: (i, 0)),
                pl.BlockSpec((_BH, c), lambda i: (i, 0)),
            ],
            core_axis_name=("c", "s"),
            dimension_semantics=(pltpu.PARALLEL,),
        )(xr_hbm, xi_hbm, b_hbm, o_hbm.at[0], o_hbm.at[1])

    return run(x_real, x_imag, bias)


def _tc_block(xr_ref, xi_ref, b_ref, o_ref):
    xr = xr_ref[...]
    xi = xi_ref[...]
    b = b_ref[...]
    r2 = xr * xr + xi * xi
    inv_r = jax.lax.rsqrt(r2)
    scale = jnp.maximum(1.0 + b * inv_r, 0.0)
    scale = jnp.where(r2 > 0.0, scale, 1.0)
    o_ref[0, :, :] = scale * xr
    o_ref[1, :, :] = scale * xi


def _tc_modrelu(x_real, x_imag, bias):
    n, c = x_real.shape
    bn = 1024
    return pl.pallas_call(
        _tc_block,
        grid=(n // bn,),
        in_specs=[
            pl.BlockSpec((bn, c), lambda i: (i, 0)),
            pl.BlockSpec((bn, c), lambda i: (i, 0)),
            pl.BlockSpec((1, c), lambda i: (0, 0)),
        ],
        out_specs=pl.BlockSpec((2, bn, c), lambda i: (0, i, 0)),
        out_shape=jax.ShapeDtypeStruct((2, n, c), x_real.dtype),
    )(x_real, x_imag, bias)


def kernel(x_real, x_imag, bias):
    return _sc_modrelu(x_real, x_imag, bias)


# speedup vs baseline: 5.7238x; 4.7468x over previous
import jax
import jax.numpy as jnp
from jax.experimental import pallas as pl
from jax.experimental.pallas import tpu as pltpu
from jax.experimental.pallas import tpu_sc as plsc


_LANES = 16
_BH = 8


def _newton_rsqrt(r2):
    i = jax.lax.bitcast_convert_type(r2, jnp.int32)
    i = jnp.int32(0x5F3759DF) - jax.lax.shift_right_logical(i, 1)
    y = jax.lax.bitcast_convert_type(i, jnp.float32)
    half = 0.5 * r2
    for _ in range(2):
        y = y * (1.5 - half * y * y)
    return y


def _sc_body(xr_vmem, xi_vmem, b_vmem, o0_vmem, o1_vmem):
    @plsc.parallel_loop(0, xr_vmem.shape[1], step=_LANES, unroll=2)
    def _(c):
        b = b_vmem.at[pl.ds(0, 1), pl.ds(c, _LANES)][...]
        for r in range(_BH):
            slc = (pl.ds(r, 1), pl.ds(c, _LANES))
            xr = xr_vmem.at[slc][...]
            xi = xi_vmem.at[slc][...]
            r2 = xr * xr + xi * xi
            scale = jnp.maximum(1.0 + b * _newton_rsqrt(r2), 0.0)
            o0_vmem.at[slc][...] = scale * xr
            o1_vmem.at[slc][...] = scale * xi


def _sc_modrelu(x_real, x_imag, bias):
    n, c = x_real.shape
    mesh = plsc.VectorSubcoreMesh(core_axis_name="c", subcore_axis_name="s")

    @pl.kernel(
        out_type=jax.ShapeDtypeStruct((2, n, c), x_real.dtype),
        mesh=mesh,
        scratch_types=[],
    )
    def run(xr_hbm, xi_hbm, b_hbm, o_hbm):
        pltpu.emit_pipeline(
            _sc_body,
            grid=(n // _BH,),
            in_specs=[
                pl.BlockSpec((_BH, c), lambda i: (i, 0)),
                pl.BlockSpec((_BH, c), lambda i: (i, 0)),
                pl.BlockSpec((1, c), lambda i: (0, 0)),
            ],
            out_specs=[
                pl.BlockSpec((_BH, c), lambda i: (i, 0)),
                pl.BlockSpec((_BH, c), lambda i: (i, 0)),
            ],
            core_axis_name=("c", "s"),
            dimension_semantics=(pltpu.PARALLEL,),
        )(xr_hbm, xi_hbm, b_hbm, o_hbm.at[0], o_hbm.at[1])

    return run(x_real, x_imag, bias)


def _tc_block(xr_ref, xi_ref, b_ref, o_ref):
    xr = xr_ref[...]
    xi = xi_ref[...]
    b = b_ref[...]
    r2 = xr * xr + xi * xi
    inv_r = jax.lax.rsqrt(r2)
    scale = jnp.maximum(1.0 + b * inv_r, 0.0)
    scale = jnp.where(r2 > 0.0, scale, 1.0)
    o_ref[0, :, :] = scale * xr
    o_ref[1, :, :] = scale * xi


def _tc_modrelu(x_real, x_imag, bias):
    n, c = x_real.shape
    bn = 1024
    return pl.pallas_call(
        _tc_block,
        grid=(n // bn,),
        in_specs=[
            pl.BlockSpec((bn, c), lambda i: (i, 0)),
            pl.BlockSpec((bn, c), lambda i: (i, 0)),
            pl.BlockSpec((1, c), lambda i: (0, 0)),
        ],
        out_specs=pl.BlockSpec((2, bn, c), lambda i: (0, i, 0)),
        out_shape=jax.ShapeDtypeStruct((2, n, c), x_real.dtype),
    )(x_real, x_imag, bias)


def kernel(x_real, x_imag, bias):
    return _sc_modrelu(x_real, x_imag, bias)


# speedup vs baseline: 7.2499x; 1.2666x over previous
import jax
import jax.numpy as jnp
from jax.experimental import pallas as pl
from jax.experimental.pallas import tpu as pltpu
from jax.experimental.pallas import tpu_sc as plsc


_LANES = 16
_BH = 8
_NEWTON_ITERS = 1
_UNROLL = 2


def _newton_rsqrt(r2):
    i = jax.lax.bitcast_convert_type(r2, jnp.int32)
    i = jnp.int32(0x5F3759DF) - jax.lax.shift_right_logical(i, 1)
    y = jax.lax.bitcast_convert_type(i, jnp.float32)
    half = 0.5 * r2
    for _ in range(_NEWTON_ITERS):
        y = y * (1.5 - half * y * y)
    return y


def _sc_body(xr_vmem, xi_vmem, b_vmem, o0_vmem, o1_vmem):
    @plsc.parallel_loop(0, xr_vmem.shape[1], step=_LANES, unroll=_UNROLL)
    def _(c):
        b = b_vmem.at[pl.ds(0, 1), pl.ds(c, _LANES)][...]
        for r in range(_BH):
            slc = (pl.ds(r, 1), pl.ds(c, _LANES))
            xr = xr_vmem.at[slc][...]
            xi = xi_vmem.at[slc][...]
            r2 = xr * xr + xi * xi
            scale = jnp.maximum(1.0 + b * _newton_rsqrt(r2), 0.0)
            o0_vmem.at[slc][...] = scale * xr
            o1_vmem.at[slc][...] = scale * xi


def _sc_modrelu(x_real, x_imag, bias):
    n, c = x_real.shape
    mesh = plsc.VectorSubcoreMesh(core_axis_name="c", subcore_axis_name="s")

    @pl.kernel(
        out_type=jax.ShapeDtypeStruct((2, n, c), x_real.dtype),
        mesh=mesh,
        scratch_types=[],
    )
    def run(xr_hbm, xi_hbm, b_hbm, o_hbm):
        pltpu.emit_pipeline(
            _sc_body,
            grid=(n // _BH,),
            in_specs=[
                pl.BlockSpec((_BH, c), lambda i: (i, 0)),
                pl.BlockSpec((_BH, c), lambda i: (i, 0)),
                pl.BlockSpec((1, c), lambda i: (0, 0)),
            ],
            out_specs=[
                pl.BlockSpec((_BH, c), lambda i: (i, 0)),
                pl.BlockSpec((_BH, c), lambda i: (i, 0)),
            ],
            core_axis_name=("c", "s"),
            dimension_semantics=(pltpu.PARALLEL,),
        )(xr_hbm, xi_hbm, b_hbm, o_hbm.at[0], o_hbm.at[1])

    return run(x_real, x_imag, bias)


def _tc_block(xr_ref, xi_ref, b_ref, o_ref):
    xr = xr_ref[...]
    xi = xi_ref[...]
    b = b_ref[...]
    r2 = xr * xr + xi * xi
    inv_r = jax.lax.rsqrt(r2)
    scale = jnp.maximum(1.0 + b * inv_r, 0.0)
    scale = jnp.where(r2 > 0.0, scale, 1.0)
    o_ref[0, :, :] = scale * xr
    o_ref[1, :, :] = scale * xi


def _tc_modrelu(x_real, x_imag, bias):
    n, c = x_real.shape
    bn = 1024
    return pl.pallas_call(
        _tc_block,
        grid=(n // bn,),
        in_specs=[
            pl.BlockSpec((bn, c), lambda i: (i, 0)),
            pl.BlockSpec((bn, c), lambda i: (i, 0)),
            pl.BlockSpec((1, c), lambda i: (0, 0)),
        ],
        out_specs=pl.BlockSpec((2, bn, c), lambda i: (0, i, 0)),
        out_shape=jax.ShapeDtypeStruct((2, n, c), x_real.dtype),
    )(x_real, x_imag, bias)


def kernel(x_real, x_imag, bias):
    return _sc_modrelu(x_real, x_imag, bias)
